# baseline (speedup 1.0000x reference)
import functools
import jax
import jax.numpy as jnp
from jax import lax
from jax.experimental import pallas as pl
from jax.experimental.pallas import tpu as pltpu
from jax.experimental.pallas import tpu_sc as plsc

B, L, D = 4, 2048, 768
R = B * L
NW = 32
CH = L // NW
W = CH * D
VL = 16


def _sc_add(x_flat, emb_flat):
    mesh = plsc.VectorSubcoreMesh(core_axis_name="c", subcore_axis_name="s")

    @functools.partial(
        pl.kernel,
        mesh=mesh,
        out_type=jax.ShapeDtypeStruct((R * D,), jnp.float32),
        scratch_types=[
            pltpu.VMEM((W,), jnp.float32),
            pltpu.VMEM((W,), jnp.float32),
        ],
    )
    def k(x_hbm, emb_hbm, out_hbm, bufe, bufx):
        wid = lax.axis_index("s") * 2 + lax.axis_index("c")
        e0 = wid * W
        pltpu.sync_copy(emb_hbm.at[pl.ds(e0, W)], bufe)
        for b in range(B):
            base = b * (L * D) + e0
            pltpu.sync_copy(x_hbm.at[pl.ds(base, W)], bufx)

            def body(j, carry):
                o = j * VL
                plsc.addupdate(bufx.at[pl.ds(o, VL)], bufe[pl.ds(o, VL)])
                return carry

            lax.fori_loop(0, W // VL, body, 0)
            pltpu.sync_copy(bufx, out_hbm.at[pl.ds(base, W)])

    return k(x_flat, emb_flat)


def kernel(x, row_embed):
    Bx, Lx, Dx = x.shape
    out = _sc_add(x.reshape(-1), row_embed.reshape(-1))
    return out.reshape(Bx, Lx, Dx)


# speedup vs baseline: 1.3072x; 1.3072x over previous
import functools
import jax
import jax.numpy as jnp
from jax import lax
from jax.experimental import pallas as pl
from jax.experimental.pallas import tpu as pltpu
from jax.experimental.pallas import tpu_sc as plsc

B, L, D = 4, 2048, 768
R = B * L
NW = 32
CH = L // NW
W = CH * D
VL = 16


def _sc_add(x_flat, emb_flat):
    mesh = plsc.VectorSubcoreMesh(core_axis_name="c", subcore_axis_name="s")

    @functools.partial(
        pl.kernel,
        mesh=mesh,
        out_type=jax.ShapeDtypeStruct((R * D,), jnp.float32),
        scratch_types=[
            pltpu.VMEM((W,), jnp.float32),
            pltpu.VMEM((W,), jnp.float32),
        ],
    )
    def k(x_hbm, emb_hbm, out_hbm, bufe, bufx):
        wid = lax.axis_index("s") * 2 + lax.axis_index("c")
        e0 = wid * W
        pltpu.sync_copy(emb_hbm.at[pl.ds(e0, W)], bufe)
        for b in range(B):
            base = b * (L * D) + e0
            pltpu.sync_copy(x_hbm.at[pl.ds(base, W)], bufx)

            @plsc.parallel_loop(0, W, step=VL, unroll=8)
            def _add(o):
                plsc.addupdate(bufx.at[pl.ds(o, VL)], bufe[pl.ds(o, VL)])

            pltpu.sync_copy(bufx, out_hbm.at[pl.ds(base, W)])

    return k(x_flat, emb_flat)


def kernel(x, row_embed):
    Bx, Lx, Dx = x.shape
    out = _sc_add(x.reshape(-1), row_embed.reshape(-1))
    return out.reshape(Bx, Lx, Dx)


# speedup vs baseline: 2.4995x; 1.9121x over previous
import functools
import jax
import jax.numpy as jnp
from jax import lax
from jax.experimental import pallas as pl
from jax.experimental.pallas import tpu as pltpu
from jax.experimental.pallas import tpu_sc as plsc

B, L, D = 4, 2048, 768
NW = 32
CH = L // NW
VL = 16


def _sc_add(x, emb):
    mesh = plsc.VectorSubcoreMesh(core_axis_name="c", subcore_axis_name="s")

    @functools.partial(
        pl.kernel,
        mesh=mesh,
        out_type=jax.ShapeDtypeStruct((B, L, D), jnp.float32),
        scratch_types=[
            pltpu.VMEM((CH, D), jnp.float32),
            pltpu.VMEM((CH, D), jnp.float32),
        ],
    )
    def k(x_hbm, emb_hbm, out_hbm, bufe, bufx):
        wid = lax.axis_index("s") * 2 + lax.axis_index("c")
        l0 = wid * CH
        pltpu.sync_copy(emb_hbm.at[pl.ds(l0, CH)], bufe)
        for b in range(B):
            pltpu.sync_copy(x_hbm.at[b, pl.ds(l0, CH)], bufx)

            @plsc.parallel_loop(0, CH, step=1, unroll=2)
            def _add(r):
                for cc in range(0, D, VL):
                    plsc.addupdate(
                        bufx.at[r, pl.ds(cc, VL)], bufe[r, pl.ds(cc, VL)]
                    )

            pltpu.sync_copy(bufx, out_hbm.at[b, pl.ds(l0, CH)])

    return k(x, emb)


def kernel(x, row_embed):
    return _sc_add(x, row_embed)
